# baseline (speedup 1.0000x reference)
import functools

import jax
import jax.numpy as jnp
from jax import lax
from jax.experimental import pallas as pl
from jax.experimental.pallas import tpu as pltpu
from jax.experimental.pallas import tpu_sc as plsc

_CHUNK = 128


@functools.lru_cache(maxsize=None)
def _make_gather(B, V, D):
    info = plsc.get_sparse_core_info()
    NC, NS = info.num_cores, info.num_subcores
    NW = NC * NS
    assert B % (8 * NW) == 0
    b_per_w = B // NW
    assert b_per_w % _CHUNK == 0
    n_chunks = b_per_w // _CHUNK
    mesh = plsc.VectorSubcoreMesh(core_axis_name="c", subcore_axis_name="s")

    @functools.partial(
        pl.kernel,
        mesh=mesh,
        out_type=jax.ShapeDtypeStruct((B, D), jnp.float32),
        scratch_types=[
            pltpu.VMEM((b_per_w,), jnp.int32),
            pltpu.VMEM((b_per_w, D), jnp.float32),
            pltpu.SemaphoreType.DMA,
        ],
        compiler_params=pltpu.CompilerParams(use_tc_tiling_on_sc=False),
    )
    def gather(table_hbm, idx_hbm, out_hbm, idx_v, rows_v, sem):
        wid = lax.axis_index("s") * NC + lax.axis_index("c")
        base = wid * b_per_w
        pltpu.sync_copy(idx_hbm.at[pl.ds(base, b_per_w)], idx_v)
        copies = []
        for c in range(n_chunks):
            cp = pltpu.make_async_copy(
                table_hbm.at[idx_v.at[pl.ds(c * _CHUNK, _CHUNK)]],
                rows_v.at[pl.ds(c * _CHUNK, _CHUNK), :],
                sem,
            )
            cp.start()
            copies.append(cp)
        for cp in copies:
            cp.wait()
        pltpu.sync_copy(rows_v, out_hbm.at[pl.ds(base, b_per_w)])

    return gather


def kernel(indices, table):
    B = indices.shape[0]
    V, D = table.shape
    return _make_gather(B, V, D)(table, indices.astype(jnp.int32))


# speedup vs baseline: 1.0398x; 1.0398x over previous
import functools

import jax
import jax.numpy as jnp
from jax import lax
from jax.experimental import pallas as pl
from jax.experimental.pallas import tpu as pltpu
from jax.experimental.pallas import tpu_sc as plsc


@functools.lru_cache(maxsize=None)
def _make_gather(B, V, D):
    info = plsc.get_sparse_core_info()
    NC, NS = info.num_cores, info.num_subcores
    NW = NC * NS
    assert B % (8 * NW) == 0
    b_per_w = B // NW
    mesh = plsc.VectorSubcoreMesh(core_axis_name="c", subcore_axis_name="s")

    @functools.partial(
        pl.kernel,
        mesh=mesh,
        out_type=jax.ShapeDtypeStruct((B, D), jnp.float32),
        scratch_types=[
            pltpu.VMEM((b_per_w,), jnp.int32),
            pltpu.SemaphoreType.DMA,
        ],
    )
    def gather(table_hbm, idx_hbm, out_hbm, idx_v, sem):
        wid = lax.axis_index("s") * NC + lax.axis_index("c")
        base = wid * b_per_w
        pltpu.sync_copy(idx_hbm.at[pl.ds(base, b_per_w)], idx_v)

        def body(g, carry):
            r0 = g * 16
            vec = idx_v[pl.ds(r0, 16)]
            for j in range(16):
                pltpu.make_async_copy(
                    table_hbm.at[pl.ds(vec[j], 1), :],
                    out_hbm.at[pl.ds(base + r0 + j, 1), :],
                    sem,
                ).start()
            return carry

        lax.fori_loop(0, b_per_w // 16, body, 0)
        pltpu.make_async_copy(
            table_hbm.at[pl.ds(0, b_per_w), :],
            out_hbm.at[pl.ds(base, b_per_w), :],
            sem,
        ).wait()

    return gather


def kernel(indices, table):
    B = indices.shape[0]
    V, D = table.shape
    return _make_gather(B, V, D)(table, indices.astype(jnp.int32))


# speedup vs baseline: 1.7369x; 1.6704x over previous
import functools

import jax
import jax.numpy as jnp
from jax import lax
from jax.experimental import pallas as pl
from jax.experimental.pallas import tpu as pltpu
from jax.experimental.pallas import tpu_sc as plsc


@functools.lru_cache(maxsize=None)
def _make_gather(B, V, D):
    info = plsc.get_sparse_core_info()
    NC, NS = info.num_cores, info.num_subcores
    NW = NC * NS
    assert B % (8 * NW) == 0
    b_per_w = B // NW
    mesh = plsc.VectorSubcoreMesh(core_axis_name="c", subcore_axis_name="s")

    @functools.partial(
        pl.kernel,
        mesh=mesh,
        out_type=jax.ShapeDtypeStruct((B, D), jnp.float32),
        scratch_types=[
            pltpu.VMEM((b_per_w,), jnp.int32),
            pltpu.VMEM((b_per_w, D), jnp.float32),
            pltpu.SemaphoreType.DMA,
        ],
    )
    def gather(table_hbm, idx_hbm, out_hbm, idx_v, rows_v, sem):
        wid = lax.axis_index("s") * NC + lax.axis_index("c")
        base = wid * b_per_w
        pltpu.sync_copy(idx_hbm.at[pl.ds(base, b_per_w)], idx_v)

        def body(g, carry):
            r0 = g * 16
            vec = idx_v[pl.ds(r0, 16)]
            for j in range(16):
                pltpu.make_async_copy(
                    table_hbm.at[pl.ds(vec[j], 1), :],
                    rows_v.at[pl.ds(r0 + j, 1), :],
                    sem,
                ).start()
            return carry

        lax.fori_loop(0, b_per_w // 16, body, 0)
        pltpu.make_async_copy(
            table_hbm.at[pl.ds(0, b_per_w), :],
            rows_v,
            sem,
        ).wait()
        pltpu.sync_copy(rows_v, out_hbm.at[pl.ds(base, b_per_w)])

    return gather


def kernel(indices, table):
    B = indices.shape[0]
    V, D = table.shape
    return _make_gather(B, V, D)(table, indices.astype(jnp.int32))


# speedup vs baseline: 2.4642x; 1.4187x over previous
import functools

import jax
import jax.numpy as jnp
from jax import lax
from jax.experimental import pallas as pl
from jax.experimental.pallas import tpu as pltpu
from jax.experimental.pallas import tpu_sc as plsc


@functools.lru_cache(maxsize=None)
def _make_gather(B, V, D):
    info = plsc.get_sparse_core_info()
    NC, NS = info.num_cores, info.num_subcores
    NW = NC * NS
    assert B % (8 * NW) == 0
    b_per_w = B // NW
    half = b_per_w // 2
    mesh = plsc.VectorSubcoreMesh(core_axis_name="c", subcore_axis_name="s")

    @functools.partial(
        pl.kernel,
        mesh=mesh,
        out_type=jax.ShapeDtypeStruct((B * D,), jnp.float32),
        scratch_types=[
            pltpu.VMEM((b_per_w,), jnp.int32),
            pltpu.VMEM((D, 128), jnp.float32),
            pltpu.VMEM((D, 128), jnp.float32),
            pltpu.VMEM((D, 128), jnp.float32),
            pltpu.VMEM((D, 128), jnp.float32),
            pltpu.VMEM((half * D,), jnp.float32),
            pltpu.SemaphoreType.DMA,
            pltpu.SemaphoreType.DMA,
            pltpu.SemaphoreType.DMA,
            pltpu.SemaphoreType.DMA,
        ],
        compiler_params=pltpu.CompilerParams(needs_layout_passes=False),
    )
    def gather(tt_hbm, idx_hbm, out_hbm, idx_v, b0, b1, b2, b3, stage,
               s0, s1, s2, s3):
        bufs = (b0, b1, b2, b3)
        sems = (s0, s1, s2, s3)
        wid = lax.axis_index("s") * NC + lax.axis_index("c")
        base = wid * b_per_w
        pltpu.sync_copy(idx_hbm.at[pl.ds(base, b_per_w)], idx_v)
        iota16 = lax.iota(jnp.int32, 16)

        def slab_start(i, slot):
            lane0 = pl.multiple_of((i // 128) * 128, 128)
            pltpu.make_async_copy(
                tt_hbm.at[:, pl.ds(lane0, 128)], bufs[slot], sems[slot]
            ).start()

        def slab_wait(slot):
            pltpu.make_async_copy(
                tt_hbm.at[:, pl.ds(0, 128)], bufs[slot], sems[slot]
            ).wait()

        vec0 = idx_v[pl.ds(0, 16)]
        for j in range(4):
            slab_start(vec0[j], j)

        def window(g, carry):
            vecw = idx_v[pl.ds(g * 16, 16)]
            vecn = idx_v[pl.ds(((g + 1) % (b_per_w // 16)) * 16, 16)]
            for j in range(16):
                slot = j % 4
                slab_wait(slot)
                i = vecw[j]
                q = i - (i // 128) * 128
                lane = lax.broadcast(q, (16,))
                r_half = (g % (b_per_w // 32)) * 16 + j
                for k in range(D // 16):
                    vals = plsc.load_gather(
                        bufs[slot], [iota16 + (16 * k), lane]
                    )
                    stage[pl.ds(r_half * D + 16 * k, 16)] = vals
                i_next = vecw[(j + 4) % 16] if j < 12 else vecn[(j + 4) % 16]
                slab_start(i_next, slot)
            return carry

        nwin = b_per_w // 16
        lax.fori_loop(0, nwin // 2, window, 0)
        pltpu.sync_copy(stage, out_hbm.at[pl.ds(base * D, half * D)])
        lax.fori_loop(nwin // 2, nwin, window, 0)
        pltpu.sync_copy(
            stage, out_hbm.at[pl.ds(base * D + half * D, half * D)]
        )
        for j in range(4):
            slab_wait(j)

    return gather


def kernel(indices, table):
    B = indices.shape[0]
    V, D = table.shape
    flat = _make_gather(B, V, D)(table.T, indices.astype(jnp.int32))
    return flat.reshape(B, D)


# speedup vs baseline: 2.8194x; 1.1442x over previous
import functools

import jax
import jax.numpy as jnp
from jax import lax
from jax.experimental import pallas as pl
from jax.experimental.pallas import tpu as pltpu
from jax.experimental.pallas import tpu_sc as plsc

_RING = 8


@functools.lru_cache(maxsize=None)
def _make_gather(B, V, D):
    info = plsc.get_sparse_core_info()
    NC, NS = info.num_cores, info.num_subcores
    NW = NC * NS
    assert B % (8 * NW) == 0
    b_per_w = B // NW
    nwin = b_per_w // 16
    nseg = 4
    wps = nwin // nseg
    seg_items = wps * 16
    mesh = plsc.VectorSubcoreMesh(core_axis_name="c", subcore_axis_name="s")

    @functools.partial(
        pl.kernel,
        mesh=mesh,
        out_type=jax.ShapeDtypeStruct((B * D,), jnp.float32),
        scratch_types=[
            pltpu.VMEM((b_per_w,), jnp.int32),
            *[pltpu.VMEM((D, 128), jnp.float32) for _ in range(_RING)],
            pltpu.VMEM((seg_items * D,), jnp.float32),
            *[pltpu.SemaphoreType.DMA for _ in range(_RING)],
        ],
        compiler_params=pltpu.CompilerParams(needs_layout_passes=False),
    )
    def gather(tt_hbm, idx_hbm, out_hbm, idx_v, *rest):
        bufs = rest[:_RING]
        stage = rest[_RING]
        sems = rest[_RING + 1:]
        wid = lax.axis_index("s") * NC + lax.axis_index("c")
        base = wid * b_per_w
        pltpu.sync_copy(idx_hbm.at[pl.ds(base, b_per_w)], idx_v)
        iota16 = lax.iota(jnp.int32, 16)

        def slab_start(i, slot):
            lane0 = pl.multiple_of((i // 128) * 128, 128)
            pltpu.make_async_copy(
                tt_hbm.at[:, pl.ds(lane0, 128)], bufs[slot], sems[slot]
            ).start()

        def slab_wait(slot):
            pltpu.make_async_copy(
                tt_hbm.at[:, pl.ds(0, 128)], bufs[slot], sems[slot]
            ).wait()

        vec0 = idx_v[pl.ds(0, 16)]
        for j in range(_RING):
            slab_start(vec0[j], j)

        def window(g, carry):
            vecw = idx_v[pl.ds(g * 16, 16)]
            vecn = idx_v[pl.ds(((g + 1) % nwin) * 16, 16)]
            for j in range(16):
                slot = j % _RING
                slab_wait(slot)
                i = vecw[j]
                q = i - (i // 128) * 128
                lane = lax.broadcast(q, (16,))
                r_seg = (g % wps) * 16 + j
                for k in range(D // 16):
                    vals = plsc.load_gather(
                        bufs[slot], [iota16 + (16 * k), lane]
                    )
                    stage[pl.ds(r_seg * D + 16 * k, 16)] = vals
                nj = j + _RING
                i_next = vecw[nj] if nj < 16 else vecn[nj % 16]
                slab_start(i_next, slot)
            return carry

        for s in range(nseg):
            lax.fori_loop(s * wps, (s + 1) * wps, window, 0)
            pltpu.sync_copy(
                stage,
                out_hbm.at[pl.ds((base + s * seg_items) * D, seg_items * D)],
            )
        for j in range(_RING):
            slab_wait(j)

    return gather


def kernel(indices, table):
    B = indices.shape[0]
    V, D = table.shape
    flat = _make_gather(B, V, D)(table.T, indices.astype(jnp.int32))
    return flat.reshape(B, D)


# speedup vs baseline: 2.8643x; 1.0159x over previous
import functools

import jax
import jax.numpy as jnp
from jax import lax
from jax.experimental import pallas as pl
from jax.experimental.pallas import tpu as pltpu
from jax.experimental.pallas import tpu_sc as plsc

_RING = 8


@functools.lru_cache(maxsize=None)
def _make_gather(B, V, D):
    info = plsc.get_sparse_core_info()
    NC, NS = info.num_cores, info.num_subcores
    NW = NC * NS
    assert B % (8 * NW) == 0
    b_per_w = B // NW
    nwin = b_per_w // 16
    nseg = 2
    wps = nwin // nseg
    seg_items = wps * 16
    mesh = plsc.VectorSubcoreMesh(core_axis_name="c", subcore_axis_name="s")

    @functools.partial(
        pl.kernel,
        mesh=mesh,
        out_type=jax.ShapeDtypeStruct((B * D,), jnp.float32),
        scratch_types=[
            pltpu.VMEM((b_per_w,), jnp.int32),
            *[pltpu.VMEM((D, 128), jnp.float32) for _ in range(_RING)],
            pltpu.VMEM((seg_items * D,), jnp.float32),
            *[pltpu.SemaphoreType.DMA for _ in range(_RING)],
        ],
        compiler_params=pltpu.CompilerParams(needs_layout_passes=False),
    )
    def gather(tt_hbm, idx_hbm, out_hbm, idx_v, *rest):
        bufs = rest[:_RING]
        stage = rest[_RING]
        sems = rest[_RING + 1:]
        wid = lax.axis_index("s") * NC + lax.axis_index("c")
        base = wid * b_per_w
        pltpu.sync_copy(idx_hbm.at[pl.ds(base, b_per_w)], idx_v)
        iota16 = lax.iota(jnp.int32, 16)

        def slab_start(lane0_i, slot):
            lane0 = pl.multiple_of(lane0_i, 128)
            pltpu.make_async_copy(
                tt_hbm.at[:, pl.ds(lane0, 128)], bufs[slot], sems[slot]
            ).start()

        def slab_wait(slot):
            pltpu.make_async_copy(
                tt_hbm.at[:, pl.ds(0, 128)], bufs[slot], sems[slot]
            ).wait()

        vec0 = idx_v[pl.ds(0, 16)]
        lane0_0 = (vec0 // 128) * 128
        for j in range(_RING):
            slab_start(lane0_0[j], j)

        def window(g, carry):
            vecw = idx_v[pl.ds(g * 16, 16)]
            vecn = idx_v[pl.ds(((g + 1) % nwin) * 16, 16)]
            lane0_w = (vecw // 128) * 128
            lane0_n = (vecn // 128) * 128
            qw = vecw - lane0_w
            for j in range(16):
                slot = j % _RING
                slab_wait(slot)
                lane = lax.broadcast(qw[j], (16,))
                r_seg = (g % wps) * 16 + j
                for k in range(D // 16):
                    vals = plsc.load_gather(
                        bufs[slot], [iota16 + (16 * k), lane]
                    )
                    stage[pl.ds(r_seg * D + 16 * k, 16)] = vals
                nj = j + _RING
                l_next = lane0_w[nj] if nj < 16 else lane0_n[nj % 16]
                slab_start(l_next, slot)
            return carry

        for s in range(nseg):
            lax.fori_loop(s * wps, (s + 1) * wps, window, 0)
            pltpu.sync_copy(
                stage,
                out_hbm.at[pl.ds((base + s * seg_items) * D, seg_items * D)],
            )
        for j in range(_RING):
            slab_wait(j)

    return gather


def kernel(indices, table):
    B = indices.shape[0]
    V, D = table.shape
    flat = _make_gather(B, V, D)(table.T, indices.astype(jnp.int32))
    return flat.reshape(B, D)
